# baseline (speedup 1.0000x reference)
import functools

import jax
import jax.numpy as jnp
import numpy as np
from jax.experimental import pallas as pl

N_ATOMS = 10000
N_EDGES = 160000
MAX_Z = 119
NUM_RADIAL = 32
NUM_FEAT = 32
CUTOFF = 5.0

_BLK = 2000


def _dense_body(feat_ref, tpad_ref, embtpad_ref, wrad_ref, wl0_ref, wl1_ref,
                wl2_ref, out_ref):
    feat = feat_ref[...]
    d = feat[:, 0:3]
    zi = feat[:, 3:4]
    zj = feat[:, 4:5]
    B = feat.shape[0]

    r2 = jnp.sum(d * d, axis=1, keepdims=True)
    r = jnp.sqrt(r2 + 1e-12)
    u = d / (r + 1e-12)
    x = u[:, 0:1]
    y = u[:, 1:2]
    z = u[:, 2:3]
    s3 = np.float32(np.sqrt(3.0))
    Y = jnp.concatenate([
        jnp.ones_like(x), y, z, x,
        s3 * x * y, s3 * y * z, 0.5 * (3.0 * z * z - 1.0),
        s3 * x * z, 0.5 * s3 * (x * x - y * y),
    ], axis=1)

    step = np.float32(CUTOFF / (NUM_RADIAL - 1))
    centers = jax.lax.broadcasted_iota(
        jnp.int32, (B, NUM_RADIAL), 1).astype(jnp.float32) * step
    gamma = np.float32((NUM_RADIAL / CUTOFF) ** 2 * 0.1)
    rbf = jnp.exp(-gamma * (r - centers) ** 2)
    env = 0.5 * (jnp.cos(np.float32(np.pi) *
                         jnp.clip(r / CUTOFF, 0.0, 1.0)) + 1.0)

    lanes = jax.lax.broadcasted_iota(
        jnp.int32, (B, 128), 1).astype(jnp.float32)
    oh_j = (lanes == zj).astype(jnp.float32)
    oh_i = (lanes == zi).astype(jnp.float32)
    emb_j = jnp.dot(oh_j, tpad_ref[...],
                    preferred_element_type=jnp.float32)
    scale = jnp.dot(oh_i, embtpad_ref[...],
                    preferred_element_type=jnp.float32)

    coeff = jnp.dot(rbf * emb_j, wrad_ref[...],
                    preferred_element_type=jnp.float32) * env
    c0 = jnp.dot(coeff, wl0_ref[...], preferred_element_type=jnp.float32)
    gate = c0 * jax.nn.sigmoid(c0)
    b1 = jnp.dot(coeff, wl1_ref[...], preferred_element_type=jnp.float32) * gate
    b2 = jnp.dot(coeff, wl2_ref[...], preferred_element_type=jnp.float32) * gate
    a = c0 * scale
    b1 = b1 * scale
    b2 = b2 * scale

    pad = jnp.zeros((B, 128 - 96 - 9), jnp.float32)
    out_ref[...] = jnp.concatenate([a, b1, b2, Y, pad], axis=1)


def _dense_stage(feat, table_pad, embt_pad, W_rad, W_l0, W_l1, W_l2):
    nblk = N_EDGES // _BLK
    rep = lambda s: pl.BlockSpec(s, lambda i: (0, 0))
    return pl.pallas_call(
        _dense_body,
        grid=(nblk,),
        in_specs=[
            pl.BlockSpec((_BLK, 8), lambda i: (i, 0)),
            rep((128, NUM_RADIAL)),
            rep((128, NUM_FEAT)),
            rep((NUM_RADIAL, NUM_FEAT)),
            rep((NUM_FEAT, NUM_FEAT)),
            rep((NUM_FEAT, NUM_FEAT)),
            rep((NUM_FEAT, NUM_FEAT)),
        ],
        out_specs=pl.BlockSpec((_BLK, 128), lambda i: (i, 0)),
        out_shape=jax.ShapeDtypeStruct((N_EDGES, 128), jnp.float32),
    )(feat, table_pad, embt_pad, W_rad, W_l0, W_l1, W_l2)


def kernel(atomic_numbers, neighbour_displacements, neighbour_indices,
           embedding_table, W_emb, W_rad, W_l0, W_l1, W_l2):
    idx_i = neighbour_indices[0]
    idx_j = neighbour_indices[1]
    z_i = jnp.take(atomic_numbers, idx_i, axis=0).astype(jnp.float32)
    z_j = jnp.take(atomic_numbers, idx_j, axis=0).astype(jnp.float32)
    feat = jnp.concatenate([
        neighbour_displacements, z_i[:, None], z_j[:, None],
        jnp.zeros((N_EDGES, 3), jnp.float32),
    ], axis=1)

    embt = jnp.dot(embedding_table, W_emb)
    pad_rows = jnp.zeros((128 - MAX_Z, NUM_RADIAL), jnp.float32)
    table_pad = jnp.concatenate([embedding_table, pad_rows], axis=0)
    embt_pad = jnp.concatenate([embt, pad_rows], axis=0)

    payload = _dense_stage(feat, table_pad, embt_pad, W_rad, W_l0, W_l1, W_l2)
    a = payload[:, 0:32]
    b1 = payload[:, 32:64]
    b2 = payload[:, 64:96]
    Y = payload[:, 96:105]

    y = jnp.concatenate([
        a[:, None, :],
        Y[:, 1:4, None] * b1[:, None, :],
        Y[:, 4:9, None] * b2[:, None, :],
    ], axis=1)
    out = jax.ops.segment_sum(y, idx_i, num_segments=N_ATOMS)
    out = out.at[:, 0, :].add(jnp.take(embt, atomic_numbers, axis=0))
    return out[:, None, :, :]
